# baseline (speedup 1.0000x reference)
import functools

import jax
import jax.numpy as jnp
from jax import lax
from jax.experimental import pallas as pl
from jax.experimental.pallas import tpu as pltpu
from jax.experimental.pallas import tpu_sc as plsc

_DIM = 128
_K = 65536
_BATCH = 4096
_CW = 4096
_NCPW = 8

_MESH = plsc.VectorSubcoreMesh(core_axis_name="c", subcore_axis_name="s")


@functools.partial(
    pl.kernel,
    out_type=jax.ShapeDtypeStruct((_DIM, _K), jnp.float32),
    mesh=_MESH,
    scratch_types=[
        pltpu.VMEM((2, 8, _CW), jnp.float32),
        pltpu.VMEM((128, 128), jnp.float32),
        pltpu.VMEM((128, 128), jnp.float32),
        pltpu.VMEM((16,), jnp.int32),
        pltpu.SemaphoreType.DMA,
        pltpu.SemaphoreType.DMA,
        pltpu.SemaphoreType.DMA,
        pltpu.SemaphoreType.DMA,
        pltpu.SemaphoreType.DMA,
        pltpu.SemaphoreType.DMA,
        pltpu.SemaphoreType.DMA,
        pltpu.SemaphoreType.DMA,
    ],
    compiler_params=pltpu.CompilerParams(needs_layout_passes=False),
)
def _sc_update(q_hbm, keys_hbm, c0_hbm, out_hbm, bufs, kt, tt, c0v,
               si0, si1, si2, so0, so1, so2, ksem, ssem):
    w = lax.axis_index("s") * 2 + lax.axis_index("c")
    sin = (si0, si1, si2)
    sout = (so0, so1, so2)

    kin = pltpu.make_async_copy(keys_hbm.at[pl.ds(w * 128, 128), :], kt, ksem)
    kin.start()
    pltpu.sync_copy(c0_hbm, c0v)
    c0 = jnp.max(c0v[...])
    kskip = c0 // 32

    def chunk_slice(ref, c):
        n = w * _NCPW + c
        return ref.at[pl.ds((n // 16) * 8, 8), pl.ds((n % 16) * _CW, _CW)]

    def in_copy(c, b):
        return pltpu.make_async_copy(chunk_slice(q_hbm, c), bufs.at[b], sin[b])

    def out_copy(c, b):
        return pltpu.make_async_copy(bufs.at[b], chunk_slice(out_hbm, c), sout[b])

    def skip(c):
        return ((w * _NCPW + c) % 16) == kskip

    for c in range(2):
        @pl.when(jnp.logical_not(skip(c)))
        def _():
            in_copy(c, c).start()

    kin.wait()
    lane = lax.iota(jnp.int32, 16)

    def tbody(d, carry):
        dvec = jnp.full((16,), d, jnp.int32)
        for cc in range(8):
            rvec = cc * 16 + lane
            vals = plsc.load_gather(kt, [rvec, dvec])
            plsc.store_scatter(tt, [dvec, rvec], vals)
        return carry

    lax.fori_loop(0, 128, tbody, 0)

    col = (c0 + w) * 128
    for a in range(16):
        pltpu.make_async_copy(
            tt.at[pl.ds(8 * a, 8), :],
            out_hbm.at[pl.ds(8 * a, 8), pl.ds(col, 128)],
            ssem,
        ).start()

    for c in range(_NCPW):
        b = c % 2

        @pl.when(jnp.logical_not(skip(c)))
        def _():
            in_copy(c, b).wait()
            out_copy(c, b).start()

        if c + 2 < _NCPW:
            @pl.when(jnp.logical_not(skip(c)))
            def _():
                out_copy(c, b).wait()

            @pl.when(jnp.logical_not(skip(c + 2)))
            def _():
                in_copy(c + 2, b).start()
    for c in range(_NCPW - 2, _NCPW):
        @pl.when(jnp.logical_not(skip(c)))
        def _():
            out_copy(c, c % 2).wait()
    for a in range(16):
        pltpu.make_async_copy(
            tt.at[pl.ds(8 * a, 8), :],
            out_hbm.at[pl.ds(8 * a, 8), pl.ds(col, 128)],
            ssem,
        ).wait()


def kernel(keys, queue, queue_ptr):
    ptr = jnp.asarray(queue_ptr, jnp.int32)
    c0rep = jnp.full((16,), ptr // 128, jnp.int32)
    new_queue = _sc_update(queue, keys, c0rep)
    new_ptr = (ptr + _BATCH) % _K
    return new_queue, jnp.asarray(new_ptr, dtype=jnp.int64)


# speedup vs baseline: 2.3731x; 2.3731x over previous
import jax
import jax.numpy as jnp
from jax.experimental import pallas as pl
from jax.experimental.pallas import tpu as pltpu

_DIM = 128
_K = 65536
_BATCH = 4096
_BLK = 16384


def _body(ptr_ref, keys_ref, queue_ref, out_ref):
    i = pl.program_id(0)
    base = i * _BLK
    ptr = ptr_ref[0]
    for s in range(_BLK // _BATCH):
        lo = s * _BATCH
        covered = (base + lo) == ptr

        @pl.when(covered)
        def _():
            out_ref[:, pl.ds(lo, _BATCH)] = keys_ref[...].T

        @pl.when(jnp.logical_not(covered))
        def _():
            out_ref[:, pl.ds(lo, _BATCH)] = queue_ref[:, pl.ds(lo, _BATCH)]


def kernel(keys, queue, queue_ptr):
    ptr = jnp.asarray(queue_ptr, jnp.int32).reshape((1,))
    new_queue = pl.pallas_call(
        _body,
        grid=(_K // _BLK,),
        in_specs=[
            pl.BlockSpec(memory_space=pltpu.SMEM),
            pl.BlockSpec((_BATCH, _DIM), lambda i: (0, 0)),
            pl.BlockSpec((_DIM, _BLK), lambda i: (0, i)),
        ],
        out_specs=pl.BlockSpec((_DIM, _BLK), lambda i: (0, i)),
        out_shape=jax.ShapeDtypeStruct((_DIM, _K), jnp.float32),
    )(ptr, keys, queue)
    new_ptr = (jnp.asarray(queue_ptr, jnp.int32) + _BATCH) % _K
    return new_queue, jnp.asarray(new_ptr, dtype=jnp.int64)
